# baseline (speedup 1.0000x reference)
import functools

import jax
import jax.numpy as jnp
from jax import lax
from jax.experimental import pallas as pl
from jax.experimental.pallas import tpu as pltpu
from jax.experimental.pallas import tpu_sc as plsc

VOCAB = 100000
EMBED_DIM = 64
NUM_CLASSES = 4
BATCH = 16384
SEQ = 200

LANES = 16
NC, NS = 2, 16
NW = NC * NS
TOK = BATCH * SEQ
TOK_W = TOK // NW
CH = 128
KB = 16
NB = TOK_W // (CH * KB)
B_W = BATCH // NW
B_SC = BATCH // NC
V_CHUNK = 6256
V_LAST = VOCAB - (NS - 1) * V_CHUNK


def _project_body(emb_ref, w_ref, b_ref, o_ref):
    o_ref[...] = (
        jnp.dot(emb_ref[...], w_ref[...], preferred_element_type=jnp.float32)
        + b_ref[0:1, :]
    )


def _project(emb_table, wpad, bpad8):
    blk = 2000
    return pl.pallas_call(
        _project_body,
        grid=(VOCAB // blk,),
        in_specs=[
            pl.BlockSpec((blk, EMBED_DIM), lambda i: (i, 0)),
            pl.BlockSpec((EMBED_DIM, LANES), lambda i: (0, 0)),
            pl.BlockSpec((8, LANES), lambda i: (0, 0)),
        ],
        out_specs=pl.BlockSpec((blk, LANES), lambda i: (i, 0)),
        out_shape=jax.ShapeDtypeStruct((VOCAB, LANES), jnp.float32),
    )(emb_table, wpad, bpad8)


def _sc_pool(p16, ids3, pat3, zrows):
    mesh = plsc.VectorSubcoreMesh(core_axis_name="c", subcore_axis_name="s")

    @functools.partial(
        pl.kernel,
        out_type=jax.ShapeDtypeStruct((BATCH, LANES), jnp.float32),
        mesh=mesh,
        compiler_params=pltpu.CompilerParams(use_tc_tiling_on_sc=False),
        scratch_types=[
            pltpu.VMEM_SHARED((B_SC, LANES), jnp.float32),
            pltpu.VMEM((2, KB, CH), jnp.int32),
            pltpu.VMEM((2, KB, CH), jnp.int32),
            pltpu.VMEM((2, CH, LANES), jnp.float32),
            pltpu.SemaphoreType.DMA,
            pltpu.SemaphoreType.DMA,
            pltpu.SemaphoreType.DMA,
        ],
    )
    def pool(p_hbm, ids_hbm, pat_hbm, z_hbm, out_hbm,
             acc_sh, ids_v, pat_v, gbuf, sem_i, sem_g, sem_s):
        c = lax.axis_index("c")
        s = lax.axis_index("s")
        w = c * NS + s

        def fire_loads(j, b):
            pltpu.async_copy(ids_hbm.at[w, pl.ds(j * KB, KB)], ids_v.at[b],
                             sem_i)
            pltpu.async_copy(pat_hbm.at[s, pl.ds(j * KB, KB)], pat_v.at[b],
                             sem_i)

        def wait_loads(j, b):
            pltpu.make_async_copy(ids_hbm.at[w, pl.ds(j * KB, KB)],
                                  ids_v.at[b], sem_i).wait()
            pltpu.make_async_copy(pat_hbm.at[s, pl.ds(j * KB, KB)],
                                  pat_v.at[b], sem_i).wait()

        def fire_gather(b, kk, q):
            pltpu.async_copy(p_hbm.at[ids_v.at[b, kk]], gbuf.at[q], sem_g)

        def wait_gather(b, kk, q):
            pltpu.make_async_copy(p_hbm.at[ids_v.at[b, kk]], gbuf.at[q],
                                  sem_g).wait()

        def fire_scatter(b, kk, q):
            pltpu.async_copy(gbuf.at[q], acc_sh.at[pat_v.at[b, kk]], sem_s,
                             add=True)

        def wait_scatter(b, kk, q):
            pltpu.make_async_copy(gbuf.at[q], acc_sh.at[pat_v.at[b, kk]],
                                  sem_s).wait()

        pltpu.sync_copy(z_hbm, acc_sh.at[pl.ds(s * B_W, B_W)])
        fire_loads(0, 0)
        wait_loads(0, 0)
        fire_loads(1, 1)
        fire_gather(0, 0, 0)

        @pl.loop(0, NB // 2)
        def _(g):
            for b in (0, 1):
                j = 2 * g + b
                for kk in range(KB):
                    q = kk % 2
                    wait_gather(b, kk, q)
                    if kk > 0:
                        wait_scatter(b, kk - 1, 1 - q)
                    elif b == 1:
                        wait_scatter(0, KB - 1, 1)
                    else:
                        @pl.when(g >= 1)
                        def _():
                            wait_scatter(1, KB - 1, 1)

                    if kk == 0:
                        if b == 0:
                            @pl.when(g >= 1)
                            def _():
                                fire_loads(j + 1, 1)
                        else:
                            @pl.when(g < NB // 2 - 1)
                            def _():
                                fire_loads(j + 1, 0)

                    if kk < KB - 1:
                        fire_gather(b, kk + 1, 1 - q)
                    else:
                        @pl.when(j < NB - 1)
                        def _():
                            wait_loads(j + 1, 1 - b)
                            fire_gather(1 - b, 0, 0)
                    fire_scatter(b, kk, q)

        wait_scatter(1, KB - 1, 1)
        pltpu.sync_copy(acc_sh.at[pl.ds(s * B_W, B_W)],
                        out_hbm.at[pl.ds(c * B_SC + s * B_W, B_W)])

    return pool(p16, ids3, pat3, zrows)


def kernel(input_ids, emb_table, fc_w, fc_b):
    f32 = jnp.float32
    wpad = jnp.zeros((EMBED_DIM, LANES), f32)
    wpad = wpad.at[:, :NUM_CLASSES].set(fc_w.astype(f32).T / SEQ)
    bpad8 = jnp.zeros((8, LANES), f32)
    bpad8 = bpad8.at[:, :NUM_CLASSES].set(fc_b.astype(f32)[None, :] / SEQ)
    p16 = _project(emb_table, wpad, bpad8)

    ids3 = input_ids.astype(jnp.int32).reshape(NW, NB * KB, CH)
    tok = jnp.arange(TOK_W, dtype=jnp.int32) // SEQ
    pat3 = (tok[None, :]
            + jnp.arange(NS, dtype=jnp.int32)[:, None] * B_W
            ).reshape(NS, NB * KB, CH)
    zrows = jnp.zeros((B_W, LANES), f32)

    out16 = _sc_pool(p16, ids3, pat3, zrows)
    return out16[:, :NUM_CLASSES]


# speedup vs baseline: 1.7233x; 1.7233x over previous
import functools

import jax
import jax.numpy as jnp
from jax import lax
from jax.experimental import pallas as pl
from jax.experimental.pallas import tpu as pltpu
from jax.experimental.pallas import tpu_sc as plsc

VOCAB = 100000
EMBED_DIM = 64
NUM_CLASSES = 4
BATCH = 16384
SEQ = 200

LANES = 16
NC, NS = 2, 16
NW = NC * NS
TOK = BATCH * SEQ
TOK_W = TOK // NW
CH = 128
KB = 16
NB = TOK_W // (CH * KB)
B_W = BATCH // NW
B_SC = BATCH // NC
V_CHUNK = 6256
V_LAST = VOCAB - (NS - 1) * V_CHUNK


def _project_body(emb_ref, w_ref, b_ref, o_ref):
    o_ref[...] = (
        jnp.dot(emb_ref[...], w_ref[...], preferred_element_type=jnp.float32)
        + b_ref[0:1, :]
    )


def _project(emb_table, wpad, bpad8):
    blk = 2000
    return pl.pallas_call(
        _project_body,
        grid=(VOCAB // blk,),
        in_specs=[
            pl.BlockSpec((blk, EMBED_DIM), lambda i: (i, 0)),
            pl.BlockSpec((EMBED_DIM, LANES), lambda i: (0, 0)),
            pl.BlockSpec((8, LANES), lambda i: (0, 0)),
        ],
        out_specs=pl.BlockSpec((blk, LANES), lambda i: (i, 0)),
        out_shape=jax.ShapeDtypeStruct((VOCAB, LANES), jnp.float32),
    )(emb_table, wpad, bpad8)


def _sc_pool(p16, ids3, pat3, zrows):
    mesh = plsc.VectorSubcoreMesh(core_axis_name="c", subcore_axis_name="s")

    @functools.partial(
        pl.kernel,
        out_type=jax.ShapeDtypeStruct((BATCH, LANES), jnp.float32),
        mesh=mesh,
        compiler_params=pltpu.CompilerParams(use_tc_tiling_on_sc=False),
        scratch_types=[
            pltpu.VMEM_SHARED((B_SC, LANES), jnp.float32),
            pltpu.VMEM((2, KB, CH), jnp.int32),
            pltpu.VMEM((2, KB, CH), jnp.int32),
            pltpu.VMEM((8, CH, LANES), jnp.float32),
            pltpu.SemaphoreType.DMA,
            pltpu.SemaphoreType.DMA,
            pltpu.SemaphoreType.DMA,
        ],
    )
    def pool(p_hbm, ids_hbm, pat_hbm, z_hbm, out_hbm,
             acc_sh, ids_v, pat_v, gbuf, sem_i, sem_g, sem_s):
        c = lax.axis_index("c")
        s = lax.axis_index("s")
        w = c * NS + s

        def fire_loads(j, b):
            pltpu.async_copy(ids_hbm.at[w, pl.ds(j * KB, KB)], ids_v.at[b],
                             sem_i)
            pltpu.async_copy(pat_hbm.at[s, pl.ds(j * KB, KB)], pat_v.at[b],
                             sem_i)

        def wait_loads(j, b):
            pltpu.make_async_copy(ids_hbm.at[w, pl.ds(j * KB, KB)],
                                  ids_v.at[b], sem_i).wait()
            pltpu.make_async_copy(pat_hbm.at[s, pl.ds(j * KB, KB)],
                                  pat_v.at[b], sem_i).wait()

        def fire_gather(b, kk, q):
            pltpu.async_copy(p_hbm.at[ids_v.at[b, kk]], gbuf.at[q], sem_g)

        def wait_gather(b, kk, q):
            pltpu.make_async_copy(p_hbm.at[ids_v.at[b, kk]], gbuf.at[q],
                                  sem_g).wait()

        def fire_scatter(b, kk, q):
            pltpu.async_copy(gbuf.at[q], acc_sh.at[pat_v.at[b, kk]], sem_s,
                             add=True)

        def wait_scatter(b, kk, q):
            pltpu.make_async_copy(gbuf.at[q], acc_sh.at[pat_v.at[b, kk]],
                                  sem_s).wait()

        DEPTH = 4

        pltpu.sync_copy(z_hbm, acc_sh.at[pl.ds(s * B_W, B_W)])
        fire_loads(0, 0)
        wait_loads(0, 0)
        fire_loads(1, 1)
        for kk in range(DEPTH):
            fire_gather(0, kk, kk)

        @pl.loop(0, NB // 2)
        def _(g):
            for b in (0, 1):
                j = 2 * g + b
                for kk in range(KB):
                    q = kk % (2 * DEPTH)
                    wait_gather(b, kk, q)
                    if kk > 0:
                        wait_scatter(b, kk - 1, (kk - 1) % (2 * DEPTH))
                    elif b == 1:
                        wait_scatter(0, KB - 1, (KB - 1) % (2 * DEPTH))
                    else:
                        @pl.when(g >= 1)
                        def _():
                            wait_scatter(1, KB - 1, (KB - 1) % (2 * DEPTH))

                    if kk == 0:
                        if b == 0:
                            @pl.when(g >= 1)
                            def _():
                                fire_loads(j + 1, 1)
                        else:
                            @pl.when(g < NB // 2 - 1)
                            def _():
                                fire_loads(j + 1, 0)

                    if kk < KB - DEPTH:
                        fire_gather(b, kk + DEPTH, (kk + DEPTH) % (2 * DEPTH))
                    else:
                        if kk == KB - DEPTH:
                            @pl.when(j < NB - 1)
                            def _():
                                wait_loads(j + 1, 1 - b)
                        @pl.when(j < NB - 1)
                        def _():
                            fire_gather(1 - b, kk - (KB - DEPTH),
                                        (kk + DEPTH) % (2 * DEPTH))
                    fire_scatter(b, kk, q)

        wait_scatter(1, KB - 1, (KB - 1) % (2 * DEPTH))
        pltpu.sync_copy(acc_sh.at[pl.ds(s * B_W, B_W)],
                        out_hbm.at[pl.ds(c * B_SC + s * B_W, B_W)])

    return pool(p16, ids3, pat3, zrows)


def kernel(input_ids, emb_table, fc_w, fc_b):
    f32 = jnp.float32
    wpad = jnp.zeros((EMBED_DIM, LANES), f32)
    wpad = wpad.at[:, :NUM_CLASSES].set(fc_w.astype(f32).T / SEQ)
    bpad8 = jnp.zeros((8, LANES), f32)
    bpad8 = bpad8.at[:, :NUM_CLASSES].set(fc_b.astype(f32)[None, :] / SEQ)
    p16 = _project(emb_table, wpad, bpad8)

    ids3 = input_ids.astype(jnp.int32).reshape(NW, NB * KB, CH)
    tok = jnp.arange(TOK_W, dtype=jnp.int32) // SEQ
    pat3 = (tok[None, :]
            + jnp.arange(NS, dtype=jnp.int32)[:, None] * B_W
            ).reshape(NS, NB * KB, CH)
    zrows = jnp.zeros((B_W, LANES), f32)

    out16 = _sc_pool(p16, ids3, pat3, zrows)
    return out16[:, :NUM_CLASSES]


# speedup vs baseline: 1.7260x; 1.0015x over previous
import functools

import jax
import jax.numpy as jnp
from jax import lax
from jax.experimental import pallas as pl
from jax.experimental.pallas import tpu as pltpu
from jax.experimental.pallas import tpu_sc as plsc

VOCAB = 100000
EMBED_DIM = 64
NUM_CLASSES = 4
BATCH = 16384
SEQ = 200

LANES = 16
RING = 8
NC, NS = 2, 16
NW = NC * NS
TOK = BATCH * SEQ
TOK_W = TOK // NW
CH = 128
KB = 16
NB = TOK_W // (CH * KB)
B_W = BATCH // NW
B_SC = BATCH // NC
V_CHUNK = 6256
V_LAST = VOCAB - (NS - 1) * V_CHUNK


def _project_body(emb_ref, w_ref, b_ref, o_ref):
    o_ref[...] = (
        jnp.dot(emb_ref[...], w_ref[...], preferred_element_type=jnp.float32)
        + b_ref[0:1, :]
    )


def _project(emb_table, wpad, bpad8):
    blk = 2000
    return pl.pallas_call(
        _project_body,
        grid=(VOCAB // blk,),
        in_specs=[
            pl.BlockSpec((blk, EMBED_DIM), lambda i: (i, 0)),
            pl.BlockSpec((EMBED_DIM, LANES), lambda i: (0, 0)),
            pl.BlockSpec((8, LANES), lambda i: (0, 0)),
        ],
        out_specs=pl.BlockSpec((blk, LANES), lambda i: (i, 0)),
        out_shape=jax.ShapeDtypeStruct((VOCAB, LANES), jnp.float32),
    )(emb_table, wpad, bpad8)


def _sc_pool(p16, ids3, pat3, zrows):
    mesh = plsc.VectorSubcoreMesh(core_axis_name="c", subcore_axis_name="s")

    @functools.partial(
        pl.kernel,
        out_type=jax.ShapeDtypeStruct((BATCH, LANES), jnp.float32),
        mesh=mesh,
        compiler_params=pltpu.CompilerParams(use_tc_tiling_on_sc=False),
        scratch_types=[
            pltpu.VMEM_SHARED((B_SC, LANES), jnp.float32),
            pltpu.VMEM((2, KB, CH), jnp.int32),
            pltpu.VMEM((2, KB, CH), jnp.int32),
            pltpu.VMEM((RING, CH, LANES), jnp.float32),
            pltpu.SemaphoreType.DMA,
            pltpu.SemaphoreType.DMA,
            pltpu.SemaphoreType.DMA,
        ],
    )
    def pool(p_hbm, ids_hbm, pat_hbm, z_hbm, out_hbm,
             acc_sh, ids_v, pat_v, gbuf, sem_i, sem_g, sem_s):
        c = lax.axis_index("c")
        s = lax.axis_index("s")
        w = c * NS + s

        def fire_loads(j, b):
            pltpu.async_copy(ids_hbm.at[w, pl.ds(j * KB, KB)], ids_v.at[b],
                             sem_i)
            pltpu.async_copy(pat_hbm.at[s, pl.ds(j * KB, KB)], pat_v.at[b],
                             sem_i)

        def wait_loads(j, b):
            pltpu.make_async_copy(ids_hbm.at[w, pl.ds(j * KB, KB)],
                                  ids_v.at[b], sem_i).wait()
            pltpu.make_async_copy(pat_hbm.at[s, pl.ds(j * KB, KB)],
                                  pat_v.at[b], sem_i).wait()

        def fire_gather(b, kk, q):
            pltpu.async_copy(p_hbm.at[ids_v.at[b, kk]], gbuf.at[q], sem_g)

        def wait_gather(b, kk, q):
            pltpu.make_async_copy(p_hbm.at[ids_v.at[b, kk]], gbuf.at[q],
                                  sem_g).wait()

        def fire_scatter(b, kk, q):
            pltpu.async_copy(gbuf.at[q], acc_sh.at[pat_v.at[b, kk]], sem_s,
                             add=True)

        def wait_scatter(b, kk, q):
            pltpu.make_async_copy(gbuf.at[q], acc_sh.at[pat_v.at[b, kk]],
                                  sem_s).wait()

        DEPTH = 4

        pltpu.sync_copy(z_hbm, acc_sh.at[pl.ds(s * B_W, B_W)])
        fire_loads(0, 0)
        wait_loads(0, 0)
        fire_loads(1, 1)
        for kk in range(DEPTH):
            fire_gather(0, kk, kk)

        @pl.loop(0, NB // 2)
        def _(g):
            for b in (0, 1):
                j = 2 * g + b
                for kk in range(KB):
                    q = kk % RING
                    wait_gather(b, kk, q)
                    if kk > 0:
                        wait_scatter(b, kk - 1, (kk - 1) % RING)
                    elif b == 1:
                        wait_scatter(0, KB - 1, (KB - 1) % RING)
                    else:
                        @pl.when(g >= 1)
                        def _():
                            wait_scatter(1, KB - 1, (KB - 1) % RING)

                    if kk == 0:
                        if b == 0:
                            @pl.when(g >= 1)
                            def _():
                                fire_loads(j + 1, 1)
                        else:
                            @pl.when(g < NB // 2 - 1)
                            def _():
                                fire_loads(j + 1, 0)

                    if kk < KB - DEPTH:
                        fire_gather(b, kk + DEPTH, (kk + DEPTH) % RING)
                    else:
                        if kk == KB - DEPTH:
                            @pl.when(j < NB - 1)
                            def _():
                                wait_loads(j + 1, 1 - b)
                        @pl.when(j < NB - 1)
                        def _():
                            fire_gather(1 - b, kk - (KB - DEPTH),
                                        (kk + DEPTH) % RING)
                    fire_scatter(b, kk, q)

        wait_scatter(1, KB - 1, (KB - 1) % RING)
        pltpu.sync_copy(acc_sh.at[pl.ds(s * B_W, B_W)],
                        out_hbm.at[pl.ds(c * B_SC + s * B_W, B_W)])

    return pool(p16, ids3, pat3, zrows)


def kernel(input_ids, emb_table, fc_w, fc_b):
    f32 = jnp.float32
    wpad = jnp.zeros((EMBED_DIM, LANES), f32)
    wpad = wpad.at[:, :NUM_CLASSES].set(fc_w.astype(f32).T / SEQ)
    bpad8 = jnp.zeros((8, LANES), f32)
    bpad8 = bpad8.at[:, :NUM_CLASSES].set(fc_b.astype(f32)[None, :] / SEQ)
    p16 = _project(emb_table, wpad, bpad8)

    ids3 = input_ids.astype(jnp.int32).reshape(NW, NB * KB, CH)
    tok = jnp.arange(TOK_W, dtype=jnp.int32) // SEQ
    pat3 = (tok[None, :]
            + jnp.arange(NS, dtype=jnp.int32)[:, None] * B_W
            ).reshape(NS, NB * KB, CH)
    zrows = jnp.zeros((B_W, LANES), f32)

    out16 = _sc_pool(p16, ids3, pat3, zrows)
    return out16[:, :NUM_CLASSES]
